# Initial kernel scaffold; baseline (speedup 1.0000x reference)
import jax
import jax.numpy as jnp
from jax.experimental import pallas as pl


def kernel(x):
    raise NotImplementedError("write your pallas kernel here")


# speedup vs baseline: 12.7651x; 12.7651x over previous
import jax
import jax.numpy as jnp
from jax.experimental import pallas as pl
from jax.experimental.pallas import tpu as pltpu

WINDOW = 11
HALO = (WINDOW - 1) // 2
EDGE = 8
TB = 1024
B, T, C = 8, 16384, 256


def _avg_kernel(xc_ref, xl_ref, xr_ref, o_ref, scratch):
    i = pl.program_id(1)
    nt = pl.num_programs(1)
    scratch[EDGE:EDGE + TB, :] = xc_ref[0]
    scratch[0:EDGE, :] = jnp.where(i > 0, xl_ref[0], 0.0)
    scratch[EDGE + TB:EDGE + TB + EDGE, :] = jnp.where(i < nt - 1, xr_ref[0], 0.0)
    base = EDGE - HALO
    acc = scratch[base:base + TB, :]
    for k in range(1, WINDOW):
        acc = acc + scratch[base + k:base + k + TB, :]
    o_ref[0] = acc * (1.0 / WINDOW)


def kernel(x):
    nt = T // TB
    grid = (B, nt)
    rows = TB // EDGE
    in_specs = [
        pl.BlockSpec((1, TB, C), lambda b, i: (b, i, 0)),
        pl.BlockSpec((1, EDGE, C), lambda b, i: (b, jnp.maximum(i * rows - 1, 0), 0)),
        pl.BlockSpec((1, EDGE, C), lambda b, i: (b, jnp.minimum((i + 1) * rows, T // EDGE - 1), 0)),
    ]
    out_spec = pl.BlockSpec((1, TB, C), lambda b, i: (b, i, 0))
    return pl.pallas_call(
        _avg_kernel,
        grid=grid,
        in_specs=in_specs,
        out_specs=out_spec,
        out_shape=jax.ShapeDtypeStruct((B, T, C), jnp.float32),
        scratch_shapes=[pltpu.VMEM((TB + 2 * EDGE, C), jnp.float32)],
        compiler_params=pltpu.CompilerParams(
            dimension_semantics=("parallel", "arbitrary"),
        ),
    )(x, x, x)


# speedup vs baseline: 14.6193x; 1.1453x over previous
import jax
import jax.numpy as jnp
from jax.experimental import pallas as pl
from jax.experimental.pallas import tpu as pltpu

WINDOW = 11
HALO = (WINDOW - 1) // 2
EDGE = 8
TB = 1024
B, T, C = 8, 16384, 256


def _avg_kernel(xc_ref, xl_ref, xr_ref, o_ref, s0, a1, a2, a3):
    i = pl.program_id(1)
    nt = pl.num_programs(1)
    s0[EDGE:EDGE + TB, :] = xc_ref[0]
    s0[0:EDGE, :] = jnp.where(i > 0, xl_ref[0], 0.0)
    s0[EDGE + TB:EDGE + TB + EDGE, :] = jnp.where(i < nt - 1, xr_ref[0], 0.0)
    a1[0:TB + 24, :] = s0[0:TB + 24, :] + s0[1:TB + 25, :]
    a2[0:TB + 16, :] = a1[0:TB + 16, :] + a1[2:TB + 18, :]
    a3[0:TB + 8, :] = a2[0:TB + 8, :] + a2[4:TB + 12, :]
    o_ref[0] = (a3[3:TB + 3, :] + a1[11:TB + 11, :] + s0[13:TB + 13, :]) * (1.0 / WINDOW)


def kernel(x):
    nt = T // TB
    grid = (B, nt)
    rows = TB // EDGE
    in_specs = [
        pl.BlockSpec((1, TB, C), lambda b, i: (b, i, 0)),
        pl.BlockSpec((1, EDGE, C), lambda b, i: (b, jnp.maximum(i * rows - 1, 0), 0)),
        pl.BlockSpec((1, EDGE, C), lambda b, i: (b, jnp.minimum((i + 1) * rows, T // EDGE - 1), 0)),
    ]
    out_spec = pl.BlockSpec((1, TB, C), lambda b, i: (b, i, 0))
    return pl.pallas_call(
        _avg_kernel,
        grid=grid,
        in_specs=in_specs,
        out_specs=out_spec,
        out_shape=jax.ShapeDtypeStruct((B, T, C), jnp.float32),
        scratch_shapes=[
            pltpu.VMEM((TB + 32, C), jnp.float32),
            pltpu.VMEM((TB + 32, C), jnp.float32),
            pltpu.VMEM((TB + 32, C), jnp.float32),
            pltpu.VMEM((TB + 16, C), jnp.float32),
        ],
        compiler_params=pltpu.CompilerParams(
            dimension_semantics=("parallel", "arbitrary"),
        ),
    )(x, x, x)


# speedup vs baseline: 18.7019x; 1.2793x over previous
import jax
import jax.numpy as jnp
from jax.experimental import pallas as pl
from jax.experimental.pallas import tpu as pltpu

WINDOW = 11
HALO = (WINDOW - 1) // 2
EDGE = 8
TB = 2048
B, T, C = 8, 16384, 256


def _avg_kernel(xc_ref, xl_ref, xr_ref, o_ref, s0, a1, a2, a3):
    i = pl.program_id(1)
    nt = pl.num_programs(1)
    s0[EDGE:EDGE + TB, :] = xc_ref[0]
    s0[0:EDGE, :] = jnp.where(i > 0, xl_ref[0], 0.0)
    s0[EDGE + TB:EDGE + TB + EDGE, :] = jnp.where(i < nt - 1, xr_ref[0], 0.0)
    a1[0:TB + 24, :] = s0[0:TB + 24, :] + s0[1:TB + 25, :]
    a2[0:TB + 16, :] = a1[0:TB + 16, :] + a1[2:TB + 18, :]
    a3[0:TB + 8, :] = a2[0:TB + 8, :] + a2[4:TB + 12, :]
    o_ref[0] = (a3[3:TB + 3, :] + a1[11:TB + 11, :] + s0[13:TB + 13, :]) * (1.0 / WINDOW)


def kernel(x):
    nt = T // TB
    grid = (B, nt)
    rows = TB // EDGE
    in_specs = [
        pl.BlockSpec((1, TB, C), lambda b, i: (b, i, 0)),
        pl.BlockSpec((1, EDGE, C), lambda b, i: (b, jnp.maximum(i * rows - 1, 0), 0)),
        pl.BlockSpec((1, EDGE, C), lambda b, i: (b, jnp.minimum((i + 1) * rows, T // EDGE - 1), 0)),
    ]
    out_spec = pl.BlockSpec((1, TB, C), lambda b, i: (b, i, 0))
    return pl.pallas_call(
        _avg_kernel,
        grid=grid,
        in_specs=in_specs,
        out_specs=out_spec,
        out_shape=jax.ShapeDtypeStruct((B, T, C), jnp.float32),
        scratch_shapes=[
            pltpu.VMEM((TB + 32, C), jnp.float32),
            pltpu.VMEM((TB + 32, C), jnp.float32),
            pltpu.VMEM((TB + 32, C), jnp.float32),
            pltpu.VMEM((TB + 16, C), jnp.float32),
        ],
        compiler_params=pltpu.CompilerParams(
            dimension_semantics=("parallel", "arbitrary"),
        ),
    )(x, x, x)


# speedup vs baseline: 21.8283x; 1.1672x over previous
import jax
import jax.numpy as jnp
from jax.experimental import pallas as pl
from jax.experimental.pallas import tpu as pltpu

WINDOW = 11
HALO = (WINDOW - 1) // 2
EDGE = 8
TB = 4096
B, T, C = 8, 16384, 256


def _avg_kernel(xc_ref, xl_ref, xr_ref, o_ref, s0, a1, a2, a3):
    i = pl.program_id(1)
    nt = pl.num_programs(1)
    s0[EDGE:EDGE + TB, :] = xc_ref[0]
    s0[0:EDGE, :] = jnp.where(i > 0, xl_ref[0], 0.0)
    s0[EDGE + TB:EDGE + TB + EDGE, :] = jnp.where(i < nt - 1, xr_ref[0], 0.0)
    a1[0:TB + 24, :] = s0[0:TB + 24, :] + s0[1:TB + 25, :]
    a2[0:TB + 16, :] = a1[0:TB + 16, :] + a1[2:TB + 18, :]
    a3[0:TB + 8, :] = a2[0:TB + 8, :] + a2[4:TB + 12, :]
    o_ref[0] = (a3[3:TB + 3, :] + a1[11:TB + 11, :] + s0[13:TB + 13, :]) * (1.0 / WINDOW)


def kernel(x):
    nt = T // TB
    grid = (B, nt)
    rows = TB // EDGE
    in_specs = [
        pl.BlockSpec((1, TB, C), lambda b, i: (b, i, 0)),
        pl.BlockSpec((1, EDGE, C), lambda b, i: (b, jnp.maximum(i * rows - 1, 0), 0)),
        pl.BlockSpec((1, EDGE, C), lambda b, i: (b, jnp.minimum((i + 1) * rows, T // EDGE - 1), 0)),
    ]
    out_spec = pl.BlockSpec((1, TB, C), lambda b, i: (b, i, 0))
    return pl.pallas_call(
        _avg_kernel,
        grid=grid,
        in_specs=in_specs,
        out_specs=out_spec,
        out_shape=jax.ShapeDtypeStruct((B, T, C), jnp.float32),
        scratch_shapes=[
            pltpu.VMEM((TB + 32, C), jnp.float32),
            pltpu.VMEM((TB + 32, C), jnp.float32),
            pltpu.VMEM((TB + 32, C), jnp.float32),
            pltpu.VMEM((TB + 16, C), jnp.float32),
        ],
        compiler_params=pltpu.CompilerParams(
            dimension_semantics=("parallel", "arbitrary"),
        ),
    )(x, x, x)


# speedup vs baseline: 22.6395x; 1.0372x over previous
import jax
import jax.numpy as jnp
from jax.experimental import pallas as pl
from jax.experimental.pallas import tpu as pltpu

WINDOW = 11
EDGE = 8
TB = 4096
B, T, C = 8, 16384, 256


def _avg_kernel(xc_ref, xl_ref, xr_ref, o_ref, a1, a2, a3t, eh, et):
    i = pl.program_id(1)
    nt = pl.num_programs(1)
    xc = xc_ref.at[0]
    eh[0:EDGE, :] = jnp.where(i > 0, xl_ref[0], 0.0)
    eh[EDGE:2 * EDGE, :] = xc[0:EDGE, :]
    et[0:EDGE, :] = xc[TB - EDGE:TB, :]
    et[EDGE:2 * EDGE, :] = jnp.where(i < nt - 1, xr_ref[0], 0.0)
    et[2 * EDGE:3 * EDGE, :] = jnp.zeros((EDGE, C), jnp.float32)
    a1[0:EDGE, :] = eh[0:EDGE, :] + eh[1:EDGE + 1, :]
    a1[EDGE:TB, :] = xc[0:TB - EDGE, :] + xc[1:TB - EDGE + 1, :]
    a1[TB:TB + 2 * EDGE, :] = et[0:2 * EDGE, :] + et[1:2 * EDGE + 1, :]
    a2[0:TB + 16, :] = a1[0:TB + 16, :] + a1[2:TB + 18, :]
    a3t[0:TB + 8, :] = a2[0:TB + 8, :] + a2[4:TB + 12, :] + a1[8:TB + 16, :]
    o_ref[0, 0:TB - EDGE, :] = (a3t[3:TB - 5, :] + xc[5:TB - 3, :]) * (1.0 / WINDOW)
    o_ref[0, TB - EDGE:TB, :] = (a3t[TB - 5:TB + 3, :] + et[5:13, :]) * (1.0 / WINDOW)


def kernel(x):
    nt = T // TB
    grid = (B, nt)
    rows = TB // EDGE
    in_specs = [
        pl.BlockSpec((1, TB, C), lambda b, i: (b, i, 0)),
        pl.BlockSpec((1, EDGE, C), lambda b, i: (b, jnp.maximum(i * rows - 1, 0), 0)),
        pl.BlockSpec((1, EDGE, C), lambda b, i: (b, jnp.minimum((i + 1) * rows, T // EDGE - 1), 0)),
    ]
    out_spec = pl.BlockSpec((1, TB, C), lambda b, i: (b, i, 0))
    return pl.pallas_call(
        _avg_kernel,
        grid=grid,
        in_specs=in_specs,
        out_specs=out_spec,
        out_shape=jax.ShapeDtypeStruct((B, T, C), jnp.float32),
        scratch_shapes=[
            pltpu.VMEM((TB + 24, C), jnp.float32),
            pltpu.VMEM((TB + 24, C), jnp.float32),
            pltpu.VMEM((TB + 8, C), jnp.float32),
            pltpu.VMEM((2 * EDGE + 8, C), jnp.float32),
            pltpu.VMEM((3 * EDGE, C), jnp.float32),
        ],
        compiler_params=pltpu.CompilerParams(
            dimension_semantics=("parallel", "arbitrary"),
        ),
    )(x, x, x)


# speedup vs baseline: 24.3501x; 1.0756x over previous
import jax
import jax.numpy as jnp
from jax.experimental import pallas as pl
from jax.experimental.pallas import tpu as pltpu

WINDOW = 11
EDGE = 8
TB = 8192
B, T, C = 8, 16384, 256


def _avg_kernel(xc_ref, xl_ref, xr_ref, o_ref, a1, a2, a3t, eh, et):
    i = pl.program_id(1)
    nt = pl.num_programs(1)
    xc = xc_ref.at[0]
    eh[0:EDGE, :] = jnp.where(i > 0, xl_ref[0], 0.0)
    eh[EDGE:2 * EDGE, :] = xc[0:EDGE, :]
    et[0:EDGE, :] = xc[TB - EDGE:TB, :]
    et[EDGE:2 * EDGE, :] = jnp.where(i < nt - 1, xr_ref[0], 0.0)
    et[2 * EDGE:3 * EDGE, :] = jnp.zeros((EDGE, C), jnp.float32)
    a1[0:EDGE, :] = eh[0:EDGE, :] + eh[1:EDGE + 1, :]
    a1[EDGE:TB, :] = xc[0:TB - EDGE, :] + xc[1:TB - EDGE + 1, :]
    a1[TB:TB + 2 * EDGE, :] = et[0:2 * EDGE, :] + et[1:2 * EDGE + 1, :]
    a2[0:TB + 16, :] = a1[0:TB + 16, :] + a1[2:TB + 18, :]
    a3t[0:TB + 8, :] = a2[0:TB + 8, :] + a2[4:TB + 12, :] + a1[8:TB + 16, :]
    o_ref[0, 0:TB - EDGE, :] = (a3t[3:TB - 5, :] + xc[5:TB - 3, :]) * (1.0 / WINDOW)
    o_ref[0, TB - EDGE:TB, :] = (a3t[TB - 5:TB + 3, :] + et[5:13, :]) * (1.0 / WINDOW)


def kernel(x):
    nt = T // TB
    grid = (B, nt)
    rows = TB // EDGE
    in_specs = [
        pl.BlockSpec((1, TB, C), lambda b, i: (b, i, 0)),
        pl.BlockSpec((1, EDGE, C), lambda b, i: (b, jnp.maximum(i * rows - 1, 0), 0)),
        pl.BlockSpec((1, EDGE, C), lambda b, i: (b, jnp.minimum((i + 1) * rows, T // EDGE - 1), 0)),
    ]
    out_spec = pl.BlockSpec((1, TB, C), lambda b, i: (b, i, 0))
    return pl.pallas_call(
        _avg_kernel,
        grid=grid,
        in_specs=in_specs,
        out_specs=out_spec,
        out_shape=jax.ShapeDtypeStruct((B, T, C), jnp.float32),
        scratch_shapes=[
            pltpu.VMEM((TB + 24, C), jnp.float32),
            pltpu.VMEM((TB + 24, C), jnp.float32),
            pltpu.VMEM((TB + 8, C), jnp.float32),
            pltpu.VMEM((2 * EDGE + 8, C), jnp.float32),
            pltpu.VMEM((3 * EDGE, C), jnp.float32),
        ],
        compiler_params=pltpu.CompilerParams(
            dimension_semantics=("parallel", "arbitrary"),
            vmem_limit_bytes=64 * 1024 * 1024,
        ),
    )(x, x, x)
